# baseline (speedup 1.0000x reference)
import functools

import jax
import jax.numpy as jnp
from jax import lax
from jax.experimental import pallas as pl
from jax.experimental.pallas import tpu as pltpu
from jax.experimental.pallas import tpu_sc as plsc

_NEXP = 64
_TOPK = 2
_BT = 2048
_NTOK = 8192
_NC = 2
_NS = 16
_TPW = _NTOK // (_NC * _NS)
_LANES = 16


def _scores_block(x_ref, w_ref, s_ref):
    x = x_ref[...]
    w = w_ref[...]
    logits = lax.dot_general(
        x, w, (((1,), (1,)), ((), ())),
        preferred_element_type=jnp.float32)
    m = jnp.max(logits, axis=1, keepdims=True)
    e = jnp.exp(logits - m)
    s_ref[...] = e / jnp.sum(e, axis=1, keepdims=True)


def _tc_scores(x, W):
    ntok, dim = x.shape
    return pl.pallas_call(
        _scores_block,
        grid=(ntok // _BT,),
        in_specs=[
            pl.BlockSpec((_BT, dim), lambda i: (i, 0)),
            pl.BlockSpec((_NEXP, dim), lambda i: (0, 0)),
        ],
        out_specs=pl.BlockSpec((_BT, _NEXP), lambda i: (i, 0)),
        out_shape=jax.ShapeDtypeStruct((ntok, _NEXP), jnp.float32),
        compiler_params=pltpu.CompilerParams(
            dimension_semantics=("arbitrary",),
        ),
    )(x, W)


def _sc_topk_body(scores_hbm, wout_hbm, iout_hbm, sc_v, wv, iv, sem):
    c = lax.axis_index("c")
    s = lax.axis_index("s")
    wid = s * _NC + c
    base = wid * _TPW
    pltpu.async_copy(
        scores_hbm.at[pl.ds(base * _NEXP, _TPW * _NEXP)], sc_v, sem).wait()

    def group(g, carry):
        rows = g * _LANES + lax.iota(jnp.int32, _LANES)
        neg = jnp.full((_LANES,), -jnp.inf, jnp.float32)
        zero = jnp.zeros((_LANES,), jnp.int32)

        def step(e, st):
            m1, i1, m2, i2 = st
            ev = jnp.full((_LANES,), e, jnp.int32)
            v = plsc.load_gather(sc_v, [rows * _NEXP + ev])
            gt1 = v > m1
            gt2 = v > m2
            i2n = jnp.where(gt1, i1, jnp.where(gt2, ev, i2))
            m2n = jnp.where(gt1, m1, jnp.where(gt2, v, m2))
            i1n = jnp.where(gt1, ev, i1)
            m1n = jnp.where(gt1, v, m1)
            return (m1n, i1n, m2n, i2n)

        m1, i1, m2, i2 = lax.fori_loop(
            0, _NEXP, step, (neg, zero, neg, zero))

        col0 = jnp.zeros((_LANES,), jnp.int32)
        col1 = jnp.ones((_LANES,), jnp.int32)
        plsc.store_scatter(wv, [rows, col0], m1)
        plsc.store_scatter(wv, [rows, col1], m2)
        plsc.store_scatter(iv, [rows, col0], i1)
        plsc.store_scatter(iv, [rows, col1], i2)
        return carry

    lax.fori_loop(0, _TPW // _LANES, group, jnp.int32(0))

    pltpu.sync_copy(wv, wout_hbm.at[pl.ds(base, _TPW), :])
    pltpu.sync_copy(iv, iout_hbm.at[pl.ds(base, _TPW), :])


def _sc_topk(scores):
    mesh = plsc.VectorSubcoreMesh(core_axis_name="c", subcore_axis_name="s")
    flat = scores.reshape((_NTOK * _NEXP,))
    fn = functools.partial(
        pl.kernel,
        mesh=mesh,
        out_type=[
            jax.ShapeDtypeStruct((_NTOK, _TOPK), jnp.float32),
            jax.ShapeDtypeStruct((_NTOK, _TOPK), jnp.int32),
        ],
        scratch_types=[
            pltpu.VMEM((_TPW * _NEXP,), jnp.float32),
            pltpu.VMEM((_TPW, _TOPK), jnp.float32),
            pltpu.VMEM((_TPW, _TOPK), jnp.int32),
            pltpu.SemaphoreType.DMA,
        ],
        compiler_params=pltpu.CompilerParams(needs_layout_passes=False),
    )(_sc_topk_body)
    return fn(flat)


def kernel(x, W):
    scores = _tc_scores(x, W)
    weights, indices = _sc_topk(scores)
    return (weights, indices)


# speedup vs baseline: 2.0335x; 2.0335x over previous
import jax
import jax.numpy as jnp
from jax import lax
from jax.experimental import pallas as pl
from jax.experimental.pallas import tpu as pltpu

_NEXP = 64
_TOPK = 2
_CHUNK = 512
_NBUF = 5


def _gate(x_hbm, w_ref, wout_ref, iout_ref, buf, sems):
    ntok = x_hbm.shape[0]
    nchunk = ntok // _CHUNK
    w = w_ref[...]

    def start(i):
        slot = lax.rem(i, _NBUF)
        pltpu.make_async_copy(
            x_hbm.at[pl.ds(i * _CHUNK, _CHUNK), :],
            buf.at[slot],
            sems.at[slot],
        ).start()

    def wait(i):
        slot = lax.rem(i, _NBUF)
        pltpu.make_async_copy(
            x_hbm.at[pl.ds(i * _CHUNK, _CHUNK), :],
            buf.at[slot],
            sems.at[slot],
        ).wait()

    for p in range(_NBUF - 1):
        start(p)

    def body(i, carry):
        wait(i)
        nxt = i + _NBUF - 1

        @pl.when(nxt < nchunk)
        def _():
            start(nxt)

        x = buf[lax.rem(i, _NBUF)]
        logits = lax.dot_general(
            x, w, (((1,), (1,)), ((), ())),
            preferred_element_type=jnp.float32)

        ids = lax.broadcasted_iota(jnp.int32, logits.shape, 1)
        m1 = jnp.max(logits, axis=1, keepdims=True)
        denom = jnp.sum(jnp.exp(logits - m1), axis=1, keepdims=True)
        big = jnp.int32(_NEXP)
        i1 = jnp.min(jnp.where(logits == m1, ids, big), axis=1, keepdims=True)
        masked = jnp.where(ids == i1, -jnp.inf, logits)
        m2 = jnp.max(masked, axis=1, keepdims=True)
        i2 = jnp.min(jnp.where(masked == m2, ids, big), axis=1, keepdims=True)

        w1 = jnp.exp(m1 - m1) / denom
        w2 = jnp.exp(m2 - m1) / denom

        slot2 = lax.broadcasted_iota(jnp.int32, (_CHUNK, _TOPK), 1)
        wout_ref[pl.ds(i * _CHUNK, _CHUNK), :] = jnp.where(slot2 == 0, w1, w2)
        iout_ref[pl.ds(i * _CHUNK, _CHUNK), :] = jnp.where(slot2 == 0, i1, i2)
        return carry

    lax.fori_loop(0, nchunk, body, jnp.int32(0))


def kernel(x, W):
    ntok, dim = x.shape
    weights, indices = pl.pallas_call(
        _gate,
        in_specs=[
            pl.BlockSpec(memory_space=pl.ANY),
            pl.BlockSpec(memory_space=pltpu.VMEM),
        ],
        out_specs=[
            pl.BlockSpec(memory_space=pltpu.VMEM),
            pl.BlockSpec(memory_space=pltpu.VMEM),
        ],
        out_shape=[
            jax.ShapeDtypeStruct((ntok, _TOPK), jnp.float32),
            jax.ShapeDtypeStruct((ntok, _TOPK), jnp.int32),
        ],
        scratch_shapes=[
            pltpu.VMEM((_NBUF, _CHUNK, dim), jnp.float32),
            pltpu.SemaphoreType.DMA((_NBUF,)),
        ],
    )(x, W)
    return (weights, indices)


# speedup vs baseline: 2.1190x; 1.0420x over previous
import jax
import jax.numpy as jnp
from jax.experimental import pallas as pl
from jax.experimental.pallas import tpu as pltpu

_NEXP = 64
_TOPK = 2
_BT = 2048


def _gate_block(x_ref, w_ref, wout_ref, iout_ref):
    x = x_ref[...]
    w = w_ref[...]
    logits = jax.lax.dot_general(
        x, w, (((1,), (1,)), ((), ())),
        preferred_element_type=jnp.float32)

    ids = jax.lax.broadcasted_iota(
        jnp.int32, logits.shape, 1).astype(jnp.float32)
    m1 = jnp.max(logits, axis=1, keepdims=True)
    denom = jnp.sum(jnp.exp(logits - m1), axis=1, keepdims=True)
    big = jnp.float32(_NEXP)
    i1 = jnp.min(jnp.where(logits == m1, ids, big), axis=1, keepdims=True)
    masked = jnp.where(ids == i1, -jnp.inf, logits)
    m2 = jnp.max(masked, axis=1, keepdims=True)
    i2 = jnp.min(jnp.where(masked == m2, ids, big), axis=1, keepdims=True)

    w1 = jnp.exp(m1 - m1) / denom
    w2 = jnp.exp(m2 - m1) / denom

    slot = jax.lax.broadcasted_iota(jnp.int32, (x.shape[0], _TOPK), 1)
    wout_ref[...] = jnp.where(slot == 0, w1, w2)
    iout_ref[...] = jnp.where(slot == 0, i1, i2).astype(jnp.int32)


def kernel(x, W):
    ntok, dim = x.shape
    grid = (ntok // _BT,)
    weights, indices = pl.pallas_call(
        _gate_block,
        grid=grid,
        in_specs=[
            pl.BlockSpec((_BT, dim), lambda i: (i, 0)),
            pl.BlockSpec((_NEXP, dim), lambda i: (0, 0)),
        ],
        out_specs=[
            pl.BlockSpec((_BT, _TOPK), lambda i: (i, 0)),
            pl.BlockSpec((_BT, _TOPK), lambda i: (i, 0)),
        ],
        out_shape=[
            jax.ShapeDtypeStruct((ntok, _TOPK), jnp.float32),
            jax.ShapeDtypeStruct((ntok, _TOPK), jnp.int32),
        ],
        compiler_params=pltpu.CompilerParams(
            dimension_semantics=("arbitrary",),
        ),
    )(x, W)
    return (weights, indices)


# speedup vs baseline: 2.1265x; 1.0036x over previous
import jax
import jax.numpy as jnp
from jax.experimental import pallas as pl
from jax.experimental.pallas import tpu as pltpu

_NEXP = 64
_TOPK = 2
_BT = 1024


def _gate_block(x_ref, w_ref, wout_ref, iout_ref):
    x = x_ref[...]
    w = w_ref[...]
    logits = jax.lax.dot_general(
        x, w, (((1,), (1,)), ((), ())),
        preferred_element_type=jnp.float32)

    ids = jax.lax.broadcasted_iota(
        jnp.int32, logits.shape, 1).astype(jnp.float32)
    m1 = jnp.max(logits, axis=1, keepdims=True)
    denom = jnp.sum(jnp.exp(logits - m1), axis=1, keepdims=True)
    big = jnp.float32(_NEXP)
    i1 = jnp.min(jnp.where(logits == m1, ids, big), axis=1, keepdims=True)
    masked = jnp.where(ids == i1, -jnp.inf, logits)
    m2 = jnp.max(masked, axis=1, keepdims=True)
    i2 = jnp.min(jnp.where(masked == m2, ids, big), axis=1, keepdims=True)

    w1 = jnp.exp(m1 - m1) / denom
    w2 = jnp.exp(m2 - m1) / denom

    slot = jax.lax.broadcasted_iota(jnp.int32, (x.shape[0], _TOPK), 1)
    wout_ref[...] = jnp.where(slot == 0, w1, w2)
    iout_ref[...] = jnp.where(slot == 0, i1, i2).astype(jnp.int32)


def kernel(x, W):
    ntok, dim = x.shape
    grid = (ntok // _BT,)
    weights, indices = pl.pallas_call(
        _gate_block,
        grid=grid,
        in_specs=[
            pl.BlockSpec((_BT, dim), lambda i: (i, 0)),
            pl.BlockSpec((_NEXP, dim), lambda i: (0, 0)),
        ],
        out_specs=[
            pl.BlockSpec((_BT, _TOPK), lambda i: (i, 0)),
            pl.BlockSpec((_BT, _TOPK), lambda i: (i, 0)),
        ],
        out_shape=[
            jax.ShapeDtypeStruct((ntok, _TOPK), jnp.float32),
            jax.ShapeDtypeStruct((ntok, _TOPK), jnp.int32),
        ],
        compiler_params=pltpu.CompilerParams(
            dimension_semantics=("arbitrary",),
        ),
    )(x, W)
    return (weights, indices)
